# Initial kernel scaffold; baseline (speedup 1.0000x reference)
import jax
import jax.numpy as jnp
from jax.experimental import pallas as pl


def kernel(x, weight):
    raise NotImplementedError("write your pallas kernel here")


# speedup vs baseline: 1.0159x; 1.0159x over previous
import functools
import math

import jax
import jax.numpy as jnp
from jax import lax
from jax.experimental import pallas as pl
from jax.experimental.pallas import tpu as pltpu
from jax.experimental.pallas import tpu_sc as plsc

D_MODEL = 32
BATCH = 16384
HIST = 50
B_TOTAL = BATCH * HIST
SCALE = math.sqrt(D_MODEL)

_info = plsc.get_sparse_core_info()
NC = _info.num_cores
NS = _info.num_subcores
NW = NC * NS
B_PER_W = B_TOTAL // NW
CHUNK = 1024
N_CHUNKS = B_PER_W // CHUNK


def _body(w_hbm, x_hbm, out_hbm, idx_v, rows_v, sem):
    wid = lax.axis_index("s") * NC + lax.axis_index("c")
    base = wid * B_PER_W

    def chunk_body(c, carry):
        off = base + c * CHUNK
        pltpu.sync_copy(x_hbm.at[pl.ds(off, CHUNK)], idx_v)
        pltpu.async_copy(w_hbm.at[idx_v], rows_v, sem).wait()

        def scale_body(r, carry2):
            rows_v[r, pl.ds(0, 16)] = rows_v[r, pl.ds(0, 16)] * SCALE
            rows_v[r, pl.ds(16, 16)] = rows_v[r, pl.ds(16, 16)] * SCALE
            return carry2

        lax.fori_loop(0, CHUNK, scale_body, 0, unroll=4)
        pltpu.sync_copy(rows_v, out_hbm.at[pl.ds(off, CHUNK)])
        return carry

    lax.fori_loop(0, N_CHUNKS, chunk_body, 0)


_sc_kernel = functools.partial(
    pl.kernel,
    out_type=jax.ShapeDtypeStruct((B_TOTAL, D_MODEL), jnp.float32),
    mesh=plsc.VectorSubcoreMesh(core_axis_name="c", subcore_axis_name="s"),
    scratch_types=[
        pltpu.VMEM((CHUNK,), jnp.int32),
        pltpu.VMEM((CHUNK, D_MODEL), jnp.float32),
        pltpu.SemaphoreType.DMA,
    ],
    compiler_params=pltpu.CompilerParams(use_tc_tiling_on_sc=False),
)(_body)


@jax.jit
def kernel(x, weight):
    flat = _sc_kernel(weight, x.reshape(-1))
    return flat.reshape(BATCH, HIST, D_MODEL)


# speedup vs baseline: 1.5552x; 1.5309x over previous
import functools
import math

import jax
import jax.numpy as jnp
from jax import lax
from jax.experimental import pallas as pl
from jax.experimental.pallas import tpu as pltpu
from jax.experimental.pallas import tpu_sc as plsc

D_MODEL = 32
BATCH = 16384
HIST = 50
SCALE = math.sqrt(D_MODEL)

_info = plsc.get_sparse_core_info()
NC = _info.num_cores
NS = _info.num_subcores
NW = NC * NS
B_PER_W = BATCH // NW
NB = 16
N_CHUNKS = B_PER_W // NB


def _body(w_hbm, x_hbm, out_hbm, xb_v, rows_v, sem):
    wid = lax.axis_index("s") * NC + lax.axis_index("c")
    b_base = wid * B_PER_W

    def chunk_body(c, carry):
        b0 = b_base + c * NB
        pltpu.sync_copy(x_hbm.at[pl.ds(b0, NB), :], xb_v)
        descs = [
            pltpu.async_copy(w_hbm.at[xb_v.at[i, :]], rows_v.at[i], sem)
            for i in range(NB)
        ]
        for d in descs:
            d.wait()

        def scale_b(bi, carry2):
            def scale_k(k, carry3):
                h = k >> 1
                off = (k & 1) * 16
                rows_v[bi, h, pl.ds(off, 16)] = (
                    rows_v[bi, h, pl.ds(off, 16)] * SCALE
                )
                return carry3

            return lax.fori_loop(0, 2 * HIST, scale_k, carry2, unroll=4)

        lax.fori_loop(0, NB, scale_b, 0)
        pltpu.sync_copy(rows_v, out_hbm.at[pl.ds(b0, NB)])
        return carry

    lax.fori_loop(0, N_CHUNKS, chunk_body, 0)


_sc_kernel = functools.partial(
    pl.kernel,
    out_type=jax.ShapeDtypeStruct((BATCH, HIST, D_MODEL), jnp.float32),
    mesh=plsc.VectorSubcoreMesh(core_axis_name="c", subcore_axis_name="s"),
    scratch_types=[
        pltpu.VMEM((NB, HIST), jnp.int32),
        pltpu.VMEM((NB, HIST, D_MODEL), jnp.float32),
        pltpu.SemaphoreType.DMA,
    ],
    compiler_params=pltpu.CompilerParams(use_tc_tiling_on_sc=False),
)(_body)


@jax.jit
def kernel(x, weight):
    return _sc_kernel(weight, x)
